# baseline (speedup 1.0000x reference)
import functools

import jax
import jax.numpy as jnp
from jax import lax
from jax.experimental import pallas as pl
from jax.experimental.pallas import tpu as pltpu
from jax.experimental.pallas import tpu_sc as plsc

N = 100000
D = 128
B = 4096
MOM = 0.1
L = 16
NC = 2
NS = 16
NW = NC * NS
SHARD = 3128
LAST_LO = (NW - 1) * SHARD
LAST_ROWS = N - LAST_LO
CH = 128
HITCAP = B + L


def _rsqrt(t):
    i = plsc.bitcast(t, jnp.int32)
    i = jnp.int32(0x5F3759DF) - (i >> 1)
    y = plsc.bitcast(i, jnp.float32)
    for _ in range(3):
        y = y * (1.5 - 0.5 * t * y * y)
    return y


@functools.partial(
    pl.kernel,
    out_type=jax.ShapeDtypeStruct((N, D), jnp.float32),
    mesh=plsc.VectorSubcoreMesh(
        core_axis_name="c", subcore_axis_name="s",
        num_cores=NC, num_subcores=NS),
    compiler_params=pltpu.CompilerParams(needs_layout_passes=False),
    scratch_types=[
        pltpu.VMEM((B,), jnp.int32),
        pltpu.VMEM((HITCAP,), jnp.int32),
        pltpu.VMEM((HITCAP,), jnp.int32),
        pltpu.VMEM((CH,), jnp.int32),
        pltpu.VMEM((CH,), jnp.int32),
        pltpu.VMEM((CH, D), jnp.float32),
        pltpu.VMEM((CH, D), jnp.float32),
        pltpu.SemaphoreType.DMA,
        pltpu.SemaphoreType.DMA,
        pltpu.SemaphoreType.DMA,
    ],
)
def _mb_update(inputs_hbm, targets_hbm, features_hbm, out_hbm,
               tgt_v, hrow_v, hpos_v, cidx_v, cpos_v, xbuf_v, obuf_v,
               csem, gsem, xsem):
    wid = lax.axis_index("s") * NC + lax.axis_index("c")
    lo = pl.multiple_of(wid * SHARD, 8)
    hi = jnp.minimum(lo + SHARD, N)
    is_last = wid == NW - 1

    @pl.when(jnp.logical_not(is_last))
    def _():
        pltpu.async_copy(features_hbm.at[pl.ds(lo, SHARD)],
                         out_hbm.at[pl.ds(lo, SHARD)], csem)

    @pl.when(is_last)
    def _():
        pltpu.async_copy(features_hbm.at[pl.ds(LAST_LO, LAST_ROWS)],
                         out_hbm.at[pl.ds(LAST_LO, LAST_ROWS)], csem)

    pltpu.sync_copy(targets_hbm, tgt_v)
    lane = lax.iota(jnp.int32, L)

    def scan_body(i, cnt):
        v = tgt_v[pl.ds(i * L, L)]
        m = (v >= lo) & (v < hi)
        mi = m.astype(jnp.int32)
        slot = cnt + plsc.cumsum(mi) - 1
        plsc.store_scatter(hrow_v, [slot], v, mask=m)
        plsc.store_scatter(hpos_v, [slot], i * L + lane, mask=m)
        return cnt + jnp.sum(mi)

    cnt = lax.fori_loop(0, B // L, scan_body, jnp.int32(0))

    @pl.when(jnp.logical_not(is_last))
    def _():
        pltpu.make_async_copy(features_hbm.at[pl.ds(lo, SHARD)],
                              out_hbm.at[pl.ds(lo, SHARD)], csem).wait()

    @pl.when(is_last)
    def _():
        pltpu.make_async_copy(features_hbm.at[pl.ds(LAST_LO, LAST_ROWS)],
                              out_hbm.at[pl.ds(LAST_LO, LAST_ROWS)],
                              csem).wait()

    def chunk_body(c, _):
        for j in range(CH // L):
            lanes = jnp.minimum(c * CH + j * L + lane, cnt - 1)
            cidx_v[pl.ds(j * L, L)] = plsc.load_gather(hrow_v, [lanes])
            cpos_v[pl.ds(j * L, L)] = plsc.load_gather(hpos_v, [lanes])
        g = pltpu.async_copy(features_hbm.at[cidx_v], obuf_v, gsem)
        x = pltpu.async_copy(inputs_hbm.at[cpos_v], xbuf_v, xsem)
        g.wait()
        x.wait()

        def row_body(r, _):
            acc = jnp.zeros((L,), jnp.float32)
            for f in range(D // L):
                old = obuf_v[r, pl.ds(f * L, L)]
                xv = xbuf_v[r, pl.ds(f * L, L)]
                nv = MOM * old + (1.0 - MOM) * xv
                obuf_v[r, pl.ds(f * L, L)] = nv
                acc = acc + nv * nv
            y = _rsqrt(jnp.broadcast_to(jnp.sum(acc), (L,)))
            for f in range(D // L):
                obuf_v[r, pl.ds(f * L, L)] = obuf_v[r, pl.ds(f * L, L)] * y
            return 0

        lax.fori_loop(0, CH, row_body, 0)
        pltpu.async_copy(obuf_v, out_hbm.at[cidx_v], gsem).wait()
        return 0

    nch = (cnt + CH - 1) // CH
    lax.fori_loop(0, nch, chunk_body, 0)


def kernel(inputs, targets, features):
    return _mb_update(inputs, targets.astype(jnp.int32), features)


# speedup vs baseline: 27.6643x; 27.6643x over previous
import functools

import jax
import jax.numpy as jnp
from jax import lax
from jax.experimental import pallas as pl
from jax.experimental.pallas import tpu as pltpu
from jax.experimental.pallas import tpu_sc as plsc

N = 100000
D = 128
B = 4096
MOM = 0.1
L = 16
NC = 2
NS = 16
NW = NC * NS
BP = B // NW


def _rsqrt(t):
    i = plsc.bitcast(t, jnp.int32)
    i = jnp.int32(0x5F3759DF) - (i >> 1)
    y = plsc.bitcast(i, jnp.float32)
    for _ in range(3):
        y = y * (1.5 - 0.5 * t * y * y)
    return y


@functools.partial(
    pl.kernel,
    out_type=(),
    mesh=plsc.VectorSubcoreMesh(
        core_axis_name="c", subcore_axis_name="s",
        num_cores=NC, num_subcores=NS),
    compiler_params=pltpu.CompilerParams(needs_layout_passes=False),
    scratch_types=[
        pltpu.VMEM((BP,), jnp.int32),
        pltpu.VMEM((BP, D), jnp.float32),
        pltpu.VMEM((BP, D), jnp.float32),
        pltpu.SemaphoreType.DMA,
        pltpu.SemaphoreType.DMA,
    ],
)
def _mb_update(inputs_hbm, targets_hbm, features_hbm, bank_hbm,
               tgt_v, xbuf_v, obuf_v, gsem, xsem):
    wid = lax.axis_index("s") * NC + lax.axis_index("c")
    base = wid * BP

    pltpu.sync_copy(targets_hbm.at[pl.ds(base, BP)], tgt_v)
    g = pltpu.async_copy(features_hbm.at[tgt_v], obuf_v, gsem)
    x = pltpu.async_copy(inputs_hbm.at[pl.ds(base, BP)], xbuf_v, xsem)
    g.wait()
    x.wait()

    def row_body(r, _):
        acc = jnp.zeros((L,), jnp.float32)
        for f in range(D // L):
            old = obuf_v[r, pl.ds(f * L, L)]
            xv = xbuf_v[r, pl.ds(f * L, L)]
            nv = MOM * old + (1.0 - MOM) * xv
            obuf_v[r, pl.ds(f * L, L)] = nv
            acc = acc + nv * nv
        y = _rsqrt(jnp.broadcast_to(jnp.sum(acc), (L,)))
        for f in range(D // L):
            obuf_v[r, pl.ds(f * L, L)] = obuf_v[r, pl.ds(f * L, L)] * y
        return 0

    lax.fori_loop(0, BP, row_body, 0)
    pltpu.async_copy(obuf_v, bank_hbm.at[tgt_v], gsem).wait()


def kernel(inputs, targets, features):
    bank = jax.new_ref(features)
    _mb_update(inputs, targets.astype(jnp.int32), features, bank)
    return bank[...]
